# baseline (speedup 1.0000x reference)
import functools

import jax
import jax.numpy as jnp
from jax import lax
from jax.experimental import pallas as pl
from jax.experimental.pallas import tpu as pltpu
from jax.experimental.pallas import tpu_sc as plsc

B, P, D = 16, 4096, 128
INTER, POOL, OUT = 64, 64, 32
K = 16
GB = 2
NSEG = B * K
W = 80
NW = 32
RPW = (B * P) // NW
CH = 256
SUB = 128

_SLOPE = 0.01


def _leaky(v):
    return jnp.maximum(v, _SLOPE * v)


def _front_body(x_ref, w1, b1, w2, b2, cen, assign_ref, keys_ref, xc_ref):
    g = pl.program_id(0)
    c = cen[...]
    cn = jnp.sum(c * c, axis=1, keepdims=True)

    PP = GB * P
    xb = x_ref[...].reshape(PP, D)
    h = _leaky(jnp.dot(xb, w1[...], preferred_element_type=jnp.float32) + b1[...])
    xc = jnp.dot(h, w2[...], preferred_element_type=jnp.float32) + b2[...]
    cross2 = lax.dot_general(-2.0 * c, xc, (((1,), (1,)), ((), ())),
                             preferred_element_type=jnp.float32)
    score = cn + cross2
    best = jnp.min(score, axis=0, keepdims=True)
    ii = lax.broadcasted_iota(jnp.int32, (K, PP), 0)
    a2d = jnp.min(jnp.where(score == best, ii, K), axis=0, keepdims=True)

    xc_ref[:, :INTER] = xc
    xc_ref[:, INTER:INTER + 1] = jnp.ones((PP, 1), jnp.float32)
    for bi in range(GB):
        sl = a2d[:, bi * P:(bi + 1) * P]
        assign_ref[bi] = sl
        keys_ref[bi] = sl + (g * GB + bi) * K


def _tc_front(x, W1, b1, W2, b2, centroids):
    full = lambda shape: pl.BlockSpec(shape, lambda b: (0,) * len(shape))
    return pl.pallas_call(
        _front_body,
        grid=(B // GB,),
        in_specs=[
            pl.BlockSpec((GB, P, D), lambda b: (b, 0, 0)),
            full((D, 64)), full((1, 64)),
            full((64, INTER)), full((1, INTER)),
            full((K, INTER)),
        ],
        out_specs=[
            pl.BlockSpec((GB, 1, P), lambda b: (b, 0, 0)),
            pl.BlockSpec((GB, 1, P), lambda b: (b, 0, 0)),
            pl.BlockSpec((GB * P, W), lambda b: (b, 0)),
        ],
        out_shape=[
            jax.ShapeDtypeStruct((B, 1, P), jnp.int32),
            jax.ShapeDtypeStruct((B, 1, P), jnp.int32),
            jax.ShapeDtypeStruct((B * P, W), jnp.float32),
        ],
    )(x, W1, b1.reshape(1, -1), W2, b2.reshape(1, -1), centroids)


@functools.partial(
    pl.kernel,
    mesh=plsc.VectorSubcoreMesh(core_axis_name="c", subcore_axis_name="s"),
    out_type=jax.ShapeDtypeStruct((2, NSEG, W), jnp.float32),
    scratch_types=[
        pltpu.VMEM((CH, W), jnp.float32),
        pltpu.VMEM((CH, W), jnp.float32),
        pltpu.VMEM((RPW // SUB, SUB), jnp.int32),
        pltpu.VMEM_SHARED((NSEG, W), jnp.float32),
        pltpu.SemaphoreType.DMA,
        pltpu.SemaphoreType.DMA,
    ],
)
def _sc_seg(xc_hbm, keys_hbm, zero_hbm, sums_out,
            rows_a, rows_b, keys_v, acc_s, sem_r, sem_s):
    cid = lax.axis_index("c")
    sid = lax.axis_index("s")
    wid = sid * 2 + cid
    base = wid * RPW
    bufs = (rows_a, rows_b)
    nchk = RPW // CH
    subc = CH // SUB

    @pl.when(sid == 0)
    def _init():
        pltpu.sync_copy(zero_hbm, acc_s)

    pltpu.sync_copy(keys_hbm.at[pl.ds(wid * (RPW // SUB), RPW // SUB), :], keys_v)
    plsc.subcore_barrier()

    pltpu.sync_copy(xc_hbm.at[pl.ds(base, CH), :], bufs[0])
    reads = [None, None]
    pend = [[], []]
    for t in range(nchk):
        cur, nxt = t % 2, (t + 1) % 2
        for h in pend[nxt]:
            h.wait()
        pend[nxt] = []
        if t + 1 < nchk:
            reads[nxt] = pltpu.async_copy(
                xc_hbm.at[pl.ds(base + (t + 1) * CH, CH), :], bufs[nxt], sem_r)
        if t > 0:
            reads[cur].wait()
        for j in range(subc):
            jj = t * subc + j
            pend[cur].append(pltpu.async_copy(
                bufs[cur].at[pl.ds(j * SUB, SUB), :],
                acc_s.at[keys_v.at[jj]], sem_s, add=True))
    for side in pend:
        for h in side:
            h.wait()
    plsc.subcore_barrier()

    @pl.when(sid == 0)
    def _flush():
        pltpu.sync_copy(acc_s, sums_out.at[cid])


def _tail_body(sp_ref, w3, b3, w4, b4, w5, b5, w6, b6, y_ref):
    part = sp_ref[0] + sp_ref[1]
    sums = part[:, :INTER]
    cnt = part[:, INTER:INTER + 1]
    mean = sums / cnt
    h2 = _leaky(jnp.dot(mean, w3[...], preferred_element_type=jnp.float32) + b3[...])
    xsp = jnp.dot(h2, w4[...], preferred_element_type=jnp.float32) + b4[...]
    xsp = jnp.where(cnt > 0, xsp, 0.0)
    ohb = (lax.broadcasted_iota(jnp.int32, (B, NSEG), 1) // K
           == lax.broadcasted_iota(jnp.int32, (B, NSEG), 0)).astype(jnp.float32)
    p = jnp.dot(ohb, xsp, preferred_element_type=jnp.float32)
    t = _leaky(jnp.dot(p, w5[...], preferred_element_type=jnp.float32) + b5[...])
    logits = jnp.dot(t, w6[...], preferred_element_type=jnp.float32) + b6[...]
    m = jnp.max(logits, axis=-1, keepdims=True)
    lse = jnp.log(jnp.sum(jnp.exp(logits - m), axis=-1, keepdims=True)) + m
    y_ref[...] = logits - lse


def _tc_tail(sums_part, W3, b3, W4, b4, W5, b5, W6, b6):
    return pl.pallas_call(
        _tail_body,
        out_shape=jax.ShapeDtypeStruct((B, OUT), jnp.float32),
    )(sums_part, W3, b3.reshape(1, -1), W4, b4.reshape(1, -1),
      W5, b5.reshape(1, -1), W6, b6.reshape(1, -1))


def kernel(x, W1, b1, W2, b2, W3, b3, W4, b4, W5, b5, W6, b6, centroids):
    assign, keys, xc_flat = _tc_front(x, W1, b1, W2, b2, centroids)
    keys2d = keys.reshape((B * P) // SUB, SUB)
    sums_part = _sc_seg(xc_flat, keys2d, jnp.zeros((NSEG, W), jnp.float32))
    y_pred = _tc_tail(sums_part, W3, b3, W4, b4, W5, b5, W6, b6)
    return (y_pred, assign.reshape(B * P))


# speedup vs baseline: 1.0099x; 1.0099x over previous
import functools

import jax
import jax.numpy as jnp
from jax import lax
from jax.experimental import pallas as pl
from jax.experimental.pallas import tpu as pltpu
from jax.experimental.pallas import tpu_sc as plsc

B, P, D = 16, 4096, 128
INTER, POOL, OUT = 64, 64, 32
K = 16
GB = 2
NSEG = B * K
W = 65
NW = 32
RPW = (B * P) // NW
CH = 256
SUB = 128

_SLOPE = 0.01


def _leaky(v):
    return jnp.maximum(v, _SLOPE * v)


def _front_body(x_ref, w1, b1, w2, b2, cen, assign_ref, keys_ref, xc_ref):
    g = pl.program_id(0)
    c = cen[...]
    cn = jnp.sum(c * c, axis=1, keepdims=True)

    PP = GB * P
    xb = x_ref[...].reshape(PP, D)
    h = _leaky(jnp.dot(xb, w1[...], preferred_element_type=jnp.float32) + b1[...])
    xc = jnp.dot(h, w2[...], preferred_element_type=jnp.float32) + b2[...]
    cross2 = lax.dot_general(-2.0 * c, xc, (((1,), (1,)), ((), ())),
                             preferred_element_type=jnp.float32)
    score = cn + cross2
    best = jnp.min(score, axis=0, keepdims=True)
    ii = lax.broadcasted_iota(jnp.int32, (K, PP), 0)
    a2d = jnp.min(jnp.where(score == best, ii, K), axis=0, keepdims=True)

    xc_ref[:, :INTER] = xc
    xc_ref[:, INTER:INTER + 1] = jnp.ones((PP, 1), jnp.float32)
    for bi in range(GB):
        sl = a2d[:, bi * P:(bi + 1) * P]
        assign_ref[bi] = sl
        keys_ref[bi] = sl + (g * GB + bi) * K


def _tc_front(x, W1, b1, W2, b2, centroids):
    full = lambda shape: pl.BlockSpec(shape, lambda b: (0,) * len(shape))
    return pl.pallas_call(
        _front_body,
        grid=(B // GB,),
        in_specs=[
            pl.BlockSpec((GB, P, D), lambda b: (b, 0, 0)),
            full((D, 64)), full((1, 64)),
            full((64, INTER)), full((1, INTER)),
            full((K, INTER)),
        ],
        out_specs=[
            pl.BlockSpec((GB, 1, P), lambda b: (b, 0, 0)),
            pl.BlockSpec((GB, 1, P), lambda b: (b, 0, 0)),
            pl.BlockSpec((GB * P, W), lambda b: (b, 0)),
        ],
        out_shape=[
            jax.ShapeDtypeStruct((B, 1, P), jnp.int32),
            jax.ShapeDtypeStruct((B, 1, P), jnp.int32),
            jax.ShapeDtypeStruct((B * P, W), jnp.float32),
        ],
    )(x, W1, b1.reshape(1, -1), W2, b2.reshape(1, -1), centroids)


@functools.partial(
    pl.kernel,
    mesh=plsc.VectorSubcoreMesh(core_axis_name="c", subcore_axis_name="s"),
    out_type=jax.ShapeDtypeStruct((2, NSEG, W), jnp.float32),
    scratch_types=[
        pltpu.VMEM((CH, W), jnp.float32),
        pltpu.VMEM((CH, W), jnp.float32),
        pltpu.VMEM((RPW // SUB, SUB), jnp.int32),
        pltpu.VMEM_SHARED((NSEG, W), jnp.float32),
        pltpu.SemaphoreType.DMA,
        pltpu.SemaphoreType.DMA,
    ],
)
def _sc_seg(xc_hbm, keys_hbm, zero_hbm, sums_out,
            rows_a, rows_b, keys_v, acc_s, sem_r, sem_s):
    cid = lax.axis_index("c")
    sid = lax.axis_index("s")
    wid = sid * 2 + cid
    base = wid * RPW
    bufs = (rows_a, rows_b)
    nchk = RPW // CH
    subc = CH // SUB

    @pl.when(sid == 0)
    def _init():
        pltpu.sync_copy(zero_hbm, acc_s)

    pltpu.sync_copy(keys_hbm.at[pl.ds(wid * (RPW // SUB), RPW // SUB), :], keys_v)
    plsc.subcore_barrier()

    pltpu.sync_copy(xc_hbm.at[pl.ds(base, CH), :], bufs[0])
    reads = [None, None]
    pend = [[], []]
    for t in range(nchk):
        cur, nxt = t % 2, (t + 1) % 2
        for h in pend[nxt]:
            h.wait()
        pend[nxt] = []
        if t + 1 < nchk:
            reads[nxt] = pltpu.async_copy(
                xc_hbm.at[pl.ds(base + (t + 1) * CH, CH), :], bufs[nxt], sem_r)
        if t > 0:
            reads[cur].wait()
        for j in range(subc):
            jj = t * subc + j
            pend[cur].append(pltpu.async_copy(
                bufs[cur].at[pl.ds(j * SUB, SUB), :],
                acc_s.at[keys_v.at[jj]], sem_s, add=True))
    for side in pend:
        for h in side:
            h.wait()
    plsc.subcore_barrier()

    @pl.when(sid == 0)
    def _flush():
        pltpu.sync_copy(acc_s, sums_out.at[cid])


def _tail_body(sp_ref, w3, b3, w4, b4, w5, b5, w6, b6, y_ref):
    part = sp_ref[0] + sp_ref[1]
    sums = part[:, :INTER]
    cnt = part[:, INTER:INTER + 1]
    mean = sums / cnt
    h2 = _leaky(jnp.dot(mean, w3[...], preferred_element_type=jnp.float32) + b3[...])
    xsp = jnp.dot(h2, w4[...], preferred_element_type=jnp.float32) + b4[...]
    xsp = jnp.where(cnt > 0, xsp, 0.0)
    ohb = (lax.broadcasted_iota(jnp.int32, (B, NSEG), 1) // K
           == lax.broadcasted_iota(jnp.int32, (B, NSEG), 0)).astype(jnp.float32)
    p = jnp.dot(ohb, xsp, preferred_element_type=jnp.float32)
    t = _leaky(jnp.dot(p, w5[...], preferred_element_type=jnp.float32) + b5[...])
    logits = jnp.dot(t, w6[...], preferred_element_type=jnp.float32) + b6[...]
    m = jnp.max(logits, axis=-1, keepdims=True)
    lse = jnp.log(jnp.sum(jnp.exp(logits - m), axis=-1, keepdims=True)) + m
    y_ref[...] = logits - lse


def _tc_tail(sums_part, W3, b3, W4, b4, W5, b5, W6, b6):
    return pl.pallas_call(
        _tail_body,
        out_shape=jax.ShapeDtypeStruct((B, OUT), jnp.float32),
    )(sums_part, W3, b3.reshape(1, -1), W4, b4.reshape(1, -1),
      W5, b5.reshape(1, -1), W6, b6.reshape(1, -1))


def kernel(x, W1, b1, W2, b2, W3, b3, W4, b4, W5, b5, W6, b6, centroids):
    assign, keys, xc_flat = _tc_front(x, W1, b1, W2, b2, centroids)
    keys2d = keys.reshape((B * P) // SUB, SUB)
    sums_part = _sc_seg(xc_flat, keys2d, jnp.zeros((NSEG, W), jnp.float32))
    y_pred = _tc_tail(sums_part, W3, b3, W4, b4, W5, b5, W6, b6)
    return (y_pred, assign.reshape(B * P))


# speedup vs baseline: 2.7120x; 2.6853x over previous
import functools

import jax
import jax.numpy as jnp
from jax import lax
from jax.experimental import pallas as pl
from jax.experimental.pallas import tpu as pltpu

B, P, D = 16, 4096, 128
INTER, POOL, OUT = 64, 64, 32
K = 16
GB = 2

_SLOPE = 0.01


def _leaky(v):
    return jnp.maximum(v, _SLOPE * v)


def _fused_body(x_ref, w1, b1, w2, b2, w3, b3, w4, b4, w5, b5, w6, b6, cen,
                assign_ref, y_ref, seg):
    g = pl.program_id(0)
    c = cen[...]
    cn = jnp.sum(c * c, axis=1, keepdims=True)

    PP = GB * P
    xb = x_ref[...].reshape(PP, D)
    h = _leaky(jnp.dot(xb, w1[...], preferred_element_type=jnp.float32) + b1[...])
    xc = jnp.dot(h, w2[...], preferred_element_type=jnp.float32) + b2[...]
    xc1 = jnp.concatenate([xc, jnp.ones((PP, 1), jnp.float32)], axis=1)
    cross2 = lax.dot_general(-2.0 * c, xc, (((1,), (1,)), ((), ())),
                             preferred_element_type=jnp.float32)
    score = cn + cross2
    best = jnp.min(score, axis=0, keepdims=True)
    ii = lax.broadcasted_iota(jnp.int32, (K, PP), 0)
    a2d = jnp.min(jnp.where(score == best, ii, K), axis=0, keepdims=True)

    oh_t = (ii == a2d).astype(jnp.float32)
    for bi in range(GB):
        assign_ref[bi] = a2d[:, bi * P:(bi + 1) * P]
        seg[pl.ds((g * GB + bi) * K, K), :] = jnp.dot(
            oh_t[:, bi * P:(bi + 1) * P], xc1[bi * P:(bi + 1) * P],
            preferred_element_type=jnp.float32)

    @pl.when(g == B // GB - 1)
    def _tail():
        sums = seg[:, :INTER]
        cnt = seg[:, INTER:INTER + 1]
        mean = sums / cnt
        h2 = _leaky(jnp.dot(mean, w3[...], preferred_element_type=jnp.float32) + b3[...])
        xsp = jnp.dot(h2, w4[...], preferred_element_type=jnp.float32) + b4[...]
        xsp = jnp.where(cnt > 0, xsp, 0.0)
        ohb = (lax.broadcasted_iota(jnp.int32, (B, B * K), 1) // K
               == lax.broadcasted_iota(jnp.int32, (B, B * K), 0)).astype(jnp.float32)
        p = jnp.dot(ohb, xsp, preferred_element_type=jnp.float32)
        t = _leaky(jnp.dot(p, w5[...], preferred_element_type=jnp.float32) + b5[...])
        logits = jnp.dot(t, w6[...], preferred_element_type=jnp.float32) + b6[...]
        m = jnp.max(logits, axis=-1, keepdims=True)
        lse = jnp.log(jnp.sum(jnp.exp(logits - m), axis=-1, keepdims=True)) + m
        y_ref[...] = logits - lse


def kernel(x, W1, b1, W2, b2, W3, b3, W4, b4, W5, b5, W6, b6, centroids):
    full = lambda shape: pl.BlockSpec(shape, lambda b: (0,) * len(shape))
    b1r, b2r, b3r, b4r = (v.reshape(1, -1) for v in (b1, b2, b3, b4))
    b5r, b6r = b5.reshape(1, -1), b6.reshape(1, -1)

    assign, y_pred = pl.pallas_call(
        _fused_body,
        grid=(B // GB,),
        in_specs=[
            pl.BlockSpec((GB, P, D), lambda b: (b, 0, 0)),
            full((D, 64)), full((1, 64)),
            full((64, INTER)), full((1, INTER)),
            full((INTER, 64)), full((1, 64)),
            full((64, POOL)), full((1, POOL)),
            full((POOL, 64)), full((1, 64)),
            full((64, OUT)), full((1, OUT)),
            full((K, INTER)),
        ],
        out_specs=[
            pl.BlockSpec((GB, 1, P), lambda b: (b, 0, 0)),
            pl.BlockSpec((B, OUT), lambda b: (0, 0)),
        ],
        out_shape=[
            jax.ShapeDtypeStruct((B, 1, P), jnp.int32),
            jax.ShapeDtypeStruct((B, OUT), jnp.float32),
        ],
        scratch_shapes=[pltpu.VMEM((B * K, INTER + 1), jnp.float32)],
    )(x, W1, b1r, W2, b2r, W3, b3r, W4, b4r, W5, b5r, W6, b6r, centroids)

    return (y_pred, assign.reshape(B * P))
